# baseline (speedup 1.0000x reference)
import functools

import jax
import jax.numpy as jnp
from jax import lax
from jax.experimental import pallas as pl
from jax.experimental.pallas import tpu as pltpu
from jax.experimental.pallas import tpu_sc as plsc

B = 8
N = 2048
NTASK = 2 * B
NSUB = 32
QPS = NTASK * N // NSUB
GB = 8
LANES = 16


def _sc_body(q_hbm, r_hbm, out_hbm, qx, qy, qz, rx, ry, rz, rc, om):
    wid = lax.axis_index("s") * 2 + lax.axis_index("c")
    task = wid // 2
    half = wid % 2
    qoff = half * QPS

    pltpu.sync_copy(q_hbm.at[task, pl.ds(qoff, QPS)], qx)
    pltpu.sync_copy(q_hbm.at[task, pl.ds(N + qoff, QPS)], qy)
    pltpu.sync_copy(q_hbm.at[task, pl.ds(2 * N + qoff, QPS)], qz)
    pltpu.sync_copy(r_hbm.at[task, pl.ds(0, N)], rx)
    pltpu.sync_copy(r_hbm.at[task, pl.ds(N, N)], ry)
    pltpu.sync_copy(r_hbm.at[task, pl.ds(2 * N, N)], rz)

    def pre(i, _):
        xv = rx[pl.ds(i * LANES, LANES)]
        yv = ry[pl.ds(i * LANES, LANES)]
        zv = rz[pl.ds(i * LANES, LANES)]
        rc[pl.ds(i * LANES, LANES)] = xv * xv + yv * yv + zv * zv
        return 0

    lax.fori_loop(0, N // LANES, pre, 0)

    def blk_body(blk, _):
        base = blk * (GB * LANES)
        qm = []
        for g in range(GB):
            o = base + g * LANES
            qm.append((-2.0 * qx[pl.ds(o, LANES)],
                       -2.0 * qy[pl.ds(o, LANES)],
                       -2.0 * qz[pl.ds(o, LANES)]))

        def chunk(jc, acc):
            jo = jc * LANES
            xv = rx[pl.ds(jo, LANES)]
            yv = ry[pl.ds(jo, LANES)]
            zv = rz[pl.ds(jo, LANES)]
            cv = rc[pl.ds(jo, LANES)]
            acc = list(acc)
            for k in range(LANES):
                xs = xv[k]
                ys = yv[k]
                zs = zv[k]
                cs = cv[k]
                for g in range(GB):
                    gx, gy, gz = qm[g]
                    t = gx * xs + (gy * ys + (gz * zs + cs))
                    acc[g] = jnp.minimum(acc[g], t)
            return tuple(acc)

        init = tuple(jnp.full((LANES,), 1e30, jnp.float32) for _ in range(GB))
        acc = lax.fori_loop(0, N // LANES, chunk, init)

        for g in range(GB):
            o = base + g * LANES
            xv = qx[pl.ds(o, LANES)]
            yv = qy[pl.ds(o, LANES)]
            zv = qz[pl.ds(o, LANES)]
            qn = xv * xv + yv * yv + zv * zv
            om[pl.ds(o, LANES)] = qn + acc[g]
        return 0

    lax.fori_loop(0, QPS // (GB * LANES), blk_body, 0)

    pltpu.sync_copy(om, out_hbm.at[wid])


_sc_chamfer = functools.partial(
    pl.kernel,
    out_type=jax.ShapeDtypeStruct((NSUB, QPS), jnp.float32),
    mesh=plsc.VectorSubcoreMesh(core_axis_name="c", subcore_axis_name="s"),
    scratch_types=[
        pltpu.VMEM((QPS,), jnp.float32),
        pltpu.VMEM((QPS,), jnp.float32),
        pltpu.VMEM((QPS,), jnp.float32),
        pltpu.VMEM((N,), jnp.float32),
        pltpu.VMEM((N,), jnp.float32),
        pltpu.VMEM((N,), jnp.float32),
        pltpu.VMEM((N,), jnp.float32),
        pltpu.VMEM((QPS,), jnp.float32),
    ],
)(_sc_body)


def _tc_finish(m_ref, o_ref):
    o_ref[0, 0] = jnp.sum(jnp.sqrt(jnp.maximum(m_ref[...], 0.0))) * (
        1.0 / (2 * B * N))


def kernel(pcs1, pcs2):
    p1t = pcs1.transpose(0, 2, 1)
    p2t = pcs2.transpose(0, 2, 1)
    q_all = jnp.concatenate([p1t, p2t], axis=0).reshape(NTASK, 3 * N)
    r_all = jnp.concatenate([p2t, p1t], axis=0).reshape(NTASK, 3 * N)

    mins = _sc_chamfer(q_all, r_all)

    res = pl.pallas_call(
        _tc_finish,
        out_shape=jax.ShapeDtypeStruct((1, 1), jnp.float32),
        in_specs=[pl.BlockSpec((NSUB, QPS), lambda: (0, 0))],
        out_specs=pl.BlockSpec(memory_space=pltpu.SMEM),
    )(mins)
    return res[0, 0]


# speedup vs baseline: 3.7553x; 3.7553x over previous
import functools

import jax
import jax.numpy as jnp
from jax import lax
from jax.experimental import pallas as pl
from jax.experimental.pallas import tpu as pltpu
from jax.experimental.pallas import tpu_sc as plsc

B = 8
N = 2048
NTASK = 2 * B
NSUB = 32
QPS = NTASK * N // NSUB
GB = 4
LANES = 16


def _sc_body(q_hbm, r_hbm, out_hbm, qx, qy, qz, rx, ry, rz, rc, om):
    wid = lax.axis_index("s") * 2 + lax.axis_index("c")
    task = wid // 2
    half = wid % 2
    qoff = half * QPS

    pltpu.sync_copy(q_hbm.at[task, pl.ds(qoff, QPS)], qx)
    pltpu.sync_copy(q_hbm.at[task, pl.ds(N + qoff, QPS)], qy)
    pltpu.sync_copy(q_hbm.at[task, pl.ds(2 * N + qoff, QPS)], qz)
    pltpu.sync_copy(r_hbm.at[task, pl.ds(0, N)], rx)
    pltpu.sync_copy(r_hbm.at[task, pl.ds(N, N)], ry)
    pltpu.sync_copy(r_hbm.at[task, pl.ds(2 * N, N)], rz)

    def pre(i, _):
        xv = rx[pl.ds(i * LANES, LANES)]
        yv = ry[pl.ds(i * LANES, LANES)]
        zv = rz[pl.ds(i * LANES, LANES)]
        rc[pl.ds(i * LANES, LANES)] = xv * xv + yv * yv + zv * zv
        return 0

    lax.fori_loop(0, N // LANES, pre, 0)

    def blk_body(blk, _):
        base = blk * (GB * LANES)
        qm = []
        for g in range(GB):
            o = base + g * LANES
            qm.append((-2.0 * qx[pl.ds(o, LANES)],
                       -2.0 * qy[pl.ds(o, LANES)],
                       -2.0 * qz[pl.ds(o, LANES)]))

        rot1 = (lax.iota(jnp.int32, LANES) + 1) % LANES

        def chunk(jc, acc):
            jo = jc * LANES
            xv = rx[pl.ds(jo, LANES)]
            yv = ry[pl.ds(jo, LANES)]
            zv = rz[pl.ds(jo, LANES)]
            cv = rc[pl.ds(jo, LANES)]
            acc = list(acc)
            for k in range(LANES):
                if k:
                    xv = jnp.take_along_axis(xv, rot1, axis=0)
                    yv = jnp.take_along_axis(yv, rot1, axis=0)
                    zv = jnp.take_along_axis(zv, rot1, axis=0)
                    cv = jnp.take_along_axis(cv, rot1, axis=0)
                for g in range(GB):
                    gx, gy, gz = qm[g]
                    t = gx * xv + (gy * yv + (gz * zv + cv))
                    acc[g] = jnp.minimum(acc[g], t)
            return tuple(acc)

        init = tuple(jnp.full((LANES,), 1e30, jnp.float32) for _ in range(GB))
        acc = lax.fori_loop(0, N // LANES, chunk, init)

        for g in range(GB):
            o = base + g * LANES
            xv = qx[pl.ds(o, LANES)]
            yv = qy[pl.ds(o, LANES)]
            zv = qz[pl.ds(o, LANES)]
            qn = xv * xv + yv * yv + zv * zv
            om[pl.ds(o, LANES)] = qn + acc[g]
        return 0

    lax.fori_loop(0, QPS // (GB * LANES), blk_body, 0)

    pltpu.sync_copy(om, out_hbm.at[wid])


_sc_chamfer = functools.partial(
    pl.kernel,
    out_type=jax.ShapeDtypeStruct((NSUB, QPS), jnp.float32),
    mesh=plsc.VectorSubcoreMesh(core_axis_name="c", subcore_axis_name="s"),
    scratch_types=[
        pltpu.VMEM((QPS,), jnp.float32),
        pltpu.VMEM((QPS,), jnp.float32),
        pltpu.VMEM((QPS,), jnp.float32),
        pltpu.VMEM((N,), jnp.float32),
        pltpu.VMEM((N,), jnp.float32),
        pltpu.VMEM((N,), jnp.float32),
        pltpu.VMEM((N,), jnp.float32),
        pltpu.VMEM((QPS,), jnp.float32),
    ],
)(_sc_body)


def _tc_finish(m_ref, o_ref):
    o_ref[0, 0] = jnp.sum(jnp.sqrt(jnp.maximum(m_ref[...], 0.0))) * (
        1.0 / (2 * B * N))


def kernel(pcs1, pcs2):
    p1t = pcs1.transpose(0, 2, 1)
    p2t = pcs2.transpose(0, 2, 1)
    q_all = jnp.concatenate([p1t, p2t], axis=0).reshape(NTASK, 3 * N)
    r_all = jnp.concatenate([p2t, p1t], axis=0).reshape(NTASK, 3 * N)

    mins = _sc_chamfer(q_all, r_all)

    res = pl.pallas_call(
        _tc_finish,
        out_shape=jax.ShapeDtypeStruct((1, 1), jnp.float32),
        in_specs=[pl.BlockSpec((NSUB, QPS), lambda: (0, 0))],
        out_specs=pl.BlockSpec(memory_space=pltpu.SMEM),
    )(mins)
    return res[0, 0]


# speedup vs baseline: 9.8261x; 2.6166x over previous
import functools

import jax
import jax.numpy as jnp
from jax import lax
from jax.experimental import pallas as pl
from jax.experimental.pallas import tpu as pltpu
from jax.experimental.pallas import tpu_sc as plsc

B = 8
N = 2048
LANES = 16
NSUB = 32

SC_B = 0
TC_B = B - SC_B

GB = 4


if SC_B:
    NTASK_SC = 2 * SC_B
    CPT = NSUB // NTASK_SC
    QPS = N // CPT

    def _sc_body(q_hbm, r_hbm, out_hbm, qx, qy, qz, rx, ry, rz, rc, om):
        wid = lax.axis_index("s") * 2 + lax.axis_index("c")
        task = wid // CPT
        qoff = (wid % CPT) * QPS

        pltpu.sync_copy(q_hbm.at[task, pl.ds(qoff, QPS)], qx)
        pltpu.sync_copy(q_hbm.at[task, pl.ds(N + qoff, QPS)], qy)
        pltpu.sync_copy(q_hbm.at[task, pl.ds(2 * N + qoff, QPS)], qz)
        pltpu.sync_copy(r_hbm.at[task, pl.ds(0, N)], rx)
        pltpu.sync_copy(r_hbm.at[task, pl.ds(N, N)], ry)
        pltpu.sync_copy(r_hbm.at[task, pl.ds(2 * N, N)], rz)

        def pre(i, _):
            xv = rx[pl.ds(i * LANES, LANES)]
            yv = ry[pl.ds(i * LANES, LANES)]
            zv = rz[pl.ds(i * LANES, LANES)]
            rc[pl.ds(i * LANES, LANES)] = xv * xv + yv * yv + zv * zv
            return 0

        lax.fori_loop(0, N // LANES, pre, 0)

        def blk_body(blk, _):
            base = blk * (GB * LANES)
            qm = []
            for g in range(GB):
                o = base + g * LANES
                qm.append((-2.0 * qx[pl.ds(o, LANES)],
                           -2.0 * qy[pl.ds(o, LANES)],
                           -2.0 * qz[pl.ds(o, LANES)]))

            rot1 = (lax.iota(jnp.int32, LANES) + 1) % LANES

            def chunk(jc, acc):
                jo = jc * LANES
                xv = rx[pl.ds(jo, LANES)]
                yv = ry[pl.ds(jo, LANES)]
                zv = rz[pl.ds(jo, LANES)]
                cv = rc[pl.ds(jo, LANES)]
                acc = list(acc)
                for k in range(LANES):
                    if k:
                        xv = jnp.take_along_axis(xv, rot1, axis=0)
                        yv = jnp.take_along_axis(yv, rot1, axis=0)
                        zv = jnp.take_along_axis(zv, rot1, axis=0)
                        cv = jnp.take_along_axis(cv, rot1, axis=0)
                    for g in range(GB):
                        gx, gy, gz = qm[g]
                        t = gx * xv + (gy * yv + (gz * zv + cv))
                        acc[g] = jnp.minimum(acc[g], t)
                return tuple(acc)

            init = tuple(
                jnp.full((LANES,), 1e30, jnp.float32) for _ in range(GB))
            acc = lax.fori_loop(0, N // LANES, chunk, init)

            for g in range(GB):
                o = base + g * LANES
                xv = qx[pl.ds(o, LANES)]
                yv = qy[pl.ds(o, LANES)]
                zv = qz[pl.ds(o, LANES)]
                qn = xv * xv + yv * yv + zv * zv
                om[pl.ds(o, LANES)] = qn + acc[g]
            return 0

        lax.fori_loop(0, QPS // (GB * LANES), blk_body, 0)

        pltpu.sync_copy(om, out_hbm.at[wid])

    _sc_chamfer = functools.partial(
        pl.kernel,
        out_type=jax.ShapeDtypeStruct((NSUB, QPS), jnp.float32),
        mesh=plsc.VectorSubcoreMesh(core_axis_name="c", subcore_axis_name="s"),
        scratch_types=[
            pltpu.VMEM((QPS,), jnp.float32),
            pltpu.VMEM((QPS,), jnp.float32),
            pltpu.VMEM((QPS,), jnp.float32),
            pltpu.VMEM((N,), jnp.float32),
            pltpu.VMEM((N,), jnp.float32),
            pltpu.VMEM((N,), jnp.float32),
            pltpu.VMEM((N,), jnp.float32),
            pltpu.VMEM((QPS,), jnp.float32),
        ],
    )(_sc_body)


TQ = 512
NI = N // TQ


def _tc_body(q_ref, rt_ref, d1_ref, d2_ref):
    i = pl.program_id(1)
    q = q_ref[0]
    rt = rt_ref[0]
    g = jnp.dot(-2.0 * q, rt, preferred_element_type=jnp.float32,
                precision=lax.Precision.HIGHEST)
    qn = jnp.sum(q * q, axis=1)
    rn = jnp.sum(rt * rt, axis=0)
    d1_ref[0, 0, :] = jnp.min(g + rn[None, :], axis=1) + qn
    cm = jnp.min(g + qn[:, None], axis=0) + rn

    @pl.when(i == 0)
    def _():
        d2_ref[0, 0, :] = jnp.full((N,), 1e30, jnp.float32)

    d2_ref[0, 0, :] = jnp.minimum(d2_ref[0, 0, :], cm)


if TC_B:
    _tc_pair = pl.pallas_call(
        _tc_body,
        grid=(TC_B, NI),
        in_specs=[
            pl.BlockSpec((1, TQ, 8), lambda b, i: (b, i, 0)),
            pl.BlockSpec((1, 8, N), lambda b, i: (b, 0, 0)),
        ],
        out_specs=[
            pl.BlockSpec((1, 1, TQ), lambda b, i: (b * NI + i, 0, 0)),
            pl.BlockSpec((1, 1, N), lambda b, i: (b, 0, 0)),
        ],
        out_shape=[
            jax.ShapeDtypeStruct((TC_B * NI, 1, TQ), jnp.float32),
            jax.ShapeDtypeStruct((TC_B, 1, N), jnp.float32),
        ],
    )


def _tc_finish(m_ref, o_ref):
    o_ref[0, 0] = jnp.sum(jnp.sqrt(jnp.maximum(m_ref[...], 0.0))) * (
        1.0 / (2 * B * N))


def kernel(pcs1, pcs2):
    parts = []
    if TC_B:
        qp = jnp.pad(pcs1[:TC_B], ((0, 0), (0, 0), (0, 5)))
        rtp = jnp.pad(pcs2[:TC_B], ((0, 0), (0, 0), (0, 5))).transpose(0, 2, 1)
        d1, d2 = _tc_pair(qp, rtp)
        parts += [d1.reshape(-1), d2.reshape(-1)]
    if SC_B:
        p1t = pcs1[TC_B:].transpose(0, 2, 1)
        p2t = pcs2[TC_B:].transpose(0, 2, 1)
        q_all = jnp.concatenate([p1t, p2t], axis=0).reshape(NTASK_SC, 3 * N)
        r_all = jnp.concatenate([p2t, p1t], axis=0).reshape(NTASK_SC, 3 * N)
        parts.append(_sc_chamfer(q_all, r_all).reshape(-1))

    mins = jnp.concatenate(parts).reshape(2 * B * N // N, N)
    res = pl.pallas_call(
        _tc_finish,
        out_shape=jax.ShapeDtypeStruct((1, 1), jnp.float32),
        in_specs=[pl.BlockSpec(mins.shape, lambda: (0, 0))],
        out_specs=pl.BlockSpec(memory_space=pltpu.SMEM),
    )(mins)
    return res[0, 0]


# speedup vs baseline: 17.7903x; 1.8105x over previous
import functools

import jax
import jax.numpy as jnp
from jax import lax
from jax.experimental import pallas as pl
from jax.experimental.pallas import tpu as pltpu
from jax.experimental.pallas import tpu_sc as plsc

B = 8
N = 2048
LANES = 16
NSUB = 32

SC_B = 0
TC_B = B - SC_B

GB = 4


if SC_B:
    NTASK_SC = 2 * SC_B
    CPT = NSUB // NTASK_SC
    QPS = N // CPT

    def _sc_body(q_hbm, r_hbm, out_hbm, qx, qy, qz, rx, ry, rz, rc, om):
        wid = lax.axis_index("s") * 2 + lax.axis_index("c")
        task = wid // CPT
        qoff = (wid % CPT) * QPS

        pltpu.sync_copy(q_hbm.at[task, pl.ds(qoff, QPS)], qx)
        pltpu.sync_copy(q_hbm.at[task, pl.ds(N + qoff, QPS)], qy)
        pltpu.sync_copy(q_hbm.at[task, pl.ds(2 * N + qoff, QPS)], qz)
        pltpu.sync_copy(r_hbm.at[task, pl.ds(0, N)], rx)
        pltpu.sync_copy(r_hbm.at[task, pl.ds(N, N)], ry)
        pltpu.sync_copy(r_hbm.at[task, pl.ds(2 * N, N)], rz)

        def pre(i, _):
            xv = rx[pl.ds(i * LANES, LANES)]
            yv = ry[pl.ds(i * LANES, LANES)]
            zv = rz[pl.ds(i * LANES, LANES)]
            rc[pl.ds(i * LANES, LANES)] = xv * xv + yv * yv + zv * zv
            return 0

        lax.fori_loop(0, N // LANES, pre, 0)

        def blk_body(blk, _):
            base = blk * (GB * LANES)
            qm = []
            for g in range(GB):
                o = base + g * LANES
                qm.append((-2.0 * qx[pl.ds(o, LANES)],
                           -2.0 * qy[pl.ds(o, LANES)],
                           -2.0 * qz[pl.ds(o, LANES)]))

            rot1 = (lax.iota(jnp.int32, LANES) + 1) % LANES

            def chunk(jc, acc):
                jo = jc * LANES
                xv = rx[pl.ds(jo, LANES)]
                yv = ry[pl.ds(jo, LANES)]
                zv = rz[pl.ds(jo, LANES)]
                cv = rc[pl.ds(jo, LANES)]
                acc = list(acc)
                for k in range(LANES):
                    if k:
                        xv = jnp.take_along_axis(xv, rot1, axis=0)
                        yv = jnp.take_along_axis(yv, rot1, axis=0)
                        zv = jnp.take_along_axis(zv, rot1, axis=0)
                        cv = jnp.take_along_axis(cv, rot1, axis=0)
                    for g in range(GB):
                        gx, gy, gz = qm[g]
                        t = gx * xv + (gy * yv + (gz * zv + cv))
                        acc[g] = jnp.minimum(acc[g], t)
                return tuple(acc)

            init = tuple(
                jnp.full((LANES,), 1e30, jnp.float32) for _ in range(GB))
            acc = lax.fori_loop(0, N // LANES, chunk, init)

            for g in range(GB):
                o = base + g * LANES
                xv = qx[pl.ds(o, LANES)]
                yv = qy[pl.ds(o, LANES)]
                zv = qz[pl.ds(o, LANES)]
                qn = xv * xv + yv * yv + zv * zv
                om[pl.ds(o, LANES)] = qn + acc[g]
            return 0

        lax.fori_loop(0, QPS // (GB * LANES), blk_body, 0)

        pltpu.sync_copy(om, out_hbm.at[wid])

    _sc_chamfer = functools.partial(
        pl.kernel,
        out_type=jax.ShapeDtypeStruct((NSUB, QPS), jnp.float32),
        mesh=plsc.VectorSubcoreMesh(core_axis_name="c", subcore_axis_name="s"),
        scratch_types=[
            pltpu.VMEM((QPS,), jnp.float32),
            pltpu.VMEM((QPS,), jnp.float32),
            pltpu.VMEM((QPS,), jnp.float32),
            pltpu.VMEM((N,), jnp.float32),
            pltpu.VMEM((N,), jnp.float32),
            pltpu.VMEM((N,), jnp.float32),
            pltpu.VMEM((N,), jnp.float32),
            pltpu.VMEM((QPS,), jnp.float32),
        ],
    )(_sc_body)


TQ = 512
NI = N // TQ


def _tc_body(q_ref, rt_ref, d1_ref, d2_ref):
    i = pl.program_id(1)
    dx = q_ref[0, :, 0:1] - rt_ref[0, 0:1, :]
    dy = q_ref[0, :, 1:2] - rt_ref[0, 1:2, :]
    dz = q_ref[0, :, 2:3] - rt_ref[0, 2:3, :]
    d = dx * dx + dy * dy + dz * dz
    d1_ref[0, 0, :] = jnp.min(d, axis=1)
    cm = jnp.min(d, axis=0)

    @pl.when(i == 0)
    def _():
        d2_ref[0, 0, :] = jnp.full((N,), 1e30, jnp.float32)

    d2_ref[0, 0, :] = jnp.minimum(d2_ref[0, 0, :], cm)


if TC_B:
    _tc_pair = pl.pallas_call(
        _tc_body,
        grid=(TC_B, NI),
        in_specs=[
            pl.BlockSpec((1, TQ, 8), lambda b, i: (b, i, 0)),
            pl.BlockSpec((1, 8, N), lambda b, i: (b, 0, 0)),
        ],
        out_specs=[
            pl.BlockSpec((1, 1, TQ), lambda b, i: (b * NI + i, 0, 0)),
            pl.BlockSpec((1, 1, N), lambda b, i: (b, 0, 0)),
        ],
        out_shape=[
            jax.ShapeDtypeStruct((TC_B * NI, 1, TQ), jnp.float32),
            jax.ShapeDtypeStruct((TC_B, 1, N), jnp.float32),
        ],
    )


def _tc_finish(m_ref, o_ref):
    o_ref[0, 0] = jnp.sum(jnp.sqrt(jnp.maximum(m_ref[...], 0.0))) * (
        1.0 / (2 * B * N))


def kernel(pcs1, pcs2):
    parts = []
    if TC_B:
        qp = jnp.pad(pcs1[:TC_B], ((0, 0), (0, 0), (0, 5)))
        rtp = jnp.pad(pcs2[:TC_B], ((0, 0), (0, 0), (0, 5))).transpose(0, 2, 1)
        d1, d2 = _tc_pair(qp, rtp)
        parts += [d1.reshape(-1), d2.reshape(-1)]
    if SC_B:
        p1t = pcs1[TC_B:].transpose(0, 2, 1)
        p2t = pcs2[TC_B:].transpose(0, 2, 1)
        q_all = jnp.concatenate([p1t, p2t], axis=0).reshape(NTASK_SC, 3 * N)
        r_all = jnp.concatenate([p2t, p1t], axis=0).reshape(NTASK_SC, 3 * N)
        parts.append(_sc_chamfer(q_all, r_all).reshape(-1))

    mins = jnp.concatenate(parts).reshape(2 * B * N // N, N)
    res = pl.pallas_call(
        _tc_finish,
        out_shape=jax.ShapeDtypeStruct((1, 1), jnp.float32),
        in_specs=[pl.BlockSpec(mins.shape, lambda: (0, 0))],
        out_specs=pl.BlockSpec(memory_space=pltpu.SMEM),
    )(mins)
    return res[0, 0]
